# baseline (speedup 1.0000x reference)
import functools

import jax
import jax.numpy as jnp
from jax import lax
from jax.experimental import pallas as pl
from jax.experimental.pallas import tpu as pltpu
from jax.experimental.pallas import tpu_sc as plsc

_N = 10000
_E = 160000
_DIN = 256
_DH = 128
_DOUT = 256
_NP = 10240
_CH = 128
_EPT = 5120
_NCK = _EPT // _CH
_RB = 1024
_GRID = _NP // _RB

_sc_mesh = plsc.VectorSubcoreMesh(core_axis_name="c", subcore_axis_name="s")


def _deg_body(dst_hbm, out_hbm, dstv, hist):
    c = lax.axis_index("c")
    s = lax.axis_index("s")
    pltpu.sync_copy(dst_hbm.at[c, s], dstv)

    def zb(i, carry):
        hist[pl.ds(i * 16, 16)] = jnp.zeros((16,), jnp.float32)
        return carry

    lax.fori_loop(0, _NP // 16, zb, 0)
    ones = jnp.full((16,), 1.0, jnp.float32)

    def hb(i, carry):
        for k in range(_CH // 16):
            idx = dstv[i, pl.ds(k * 16, 16)]
            plsc.addupdate_scatter(hist, [idx], ones)
        return carry

    lax.fori_loop(0, _NCK, hb, 0)
    wid = c * 16 + s
    pltpu.sync_copy(hist, out_hbm.at[wid])


_deg_kernel = functools.partial(
    pl.kernel,
    out_type=jax.ShapeDtypeStruct((32, _NP), jnp.float32),
    mesh=_sc_mesh,
    compiler_params=pltpu.CompilerParams(needs_layout_passes=False),
    scratch_types=[
        pltpu.VMEM((_NCK, _CH), jnp.int32),
        pltpu.VMEM((_NP,), jnp.float32),
    ],
)(_deg_body)


def _scat_body(z_hbm, src_hbm, dst_hbm, out_hbm, srcv, dstv, g0, g1, acc, s0, s1):
    c = lax.axis_index("c")
    s = lax.axis_index("s")
    pltpu.sync_copy(src_hbm.at[c, s], srcv)
    pltpu.sync_copy(dst_hbm.at[c, s], dstv)

    def zb(i, carry):
        for k in range(_DH // 16):
            g0[i, pl.ds(k * 16, 16)] = jnp.zeros((16,), jnp.float32)
        return carry

    lax.fori_loop(0, _CH, zb, 0)
    rows_per_tile = _NP // 16
    for k in range(rows_per_tile // _CH):
        pltpu.sync_copy(g0, acc.at[pl.ds(s * rows_per_tile + k * _CH, _CH)])
    plsc.subcore_barrier()

    def start(k, buf, sem):
        pltpu.async_copy(z_hbm.at[srcv.at[k]], buf, sem)

    def wait(buf, sem):
        pltpu.make_async_copy(z_hbm.at[pl.ds(0, _CH)], buf, sem).wait()

    def scat(k, buf):
        pltpu.sync_copy(buf, acc.at[dstv.at[k]], add=True)

    start(0, g0, s0)
    start(1, g1, s1)

    def body(j, carry):
        k = j * 2
        wait(g0, s0)
        scat(k, g0)

        @pl.when(k + 2 < _NCK)
        def _():
            start(k + 2, g0, s0)

        wait(g1, s1)
        scat(k + 1, g1)

        @pl.when(k + 3 < _NCK)
        def _():
            start(k + 3, g1, s1)

        return carry

    lax.fori_loop(0, _NCK // 2, body, 0)
    plsc.subcore_barrier()

    def wb(k, carry):
        base = s * rows_per_tile + k * _CH
        pltpu.sync_copy(acc.at[pl.ds(base, _CH)], out_hbm.at[c, pl.ds(base, _CH)])
        return carry

    lax.fori_loop(0, rows_per_tile // _CH, wb, 0)


_scat_kernel = functools.partial(
    pl.kernel,
    out_type=jax.ShapeDtypeStruct((2, _NP, _DH), jnp.float32),
    mesh=_sc_mesh,
    compiler_params=pltpu.CompilerParams(use_tc_tiling_on_sc=False),
    scratch_types=[
        pltpu.VMEM((_NCK, _CH), jnp.int32),
        pltpu.VMEM((_NCK, _CH), jnp.int32),
        pltpu.VMEM((_CH, _DH), jnp.float32),
        pltpu.VMEM((_CH, _DH), jnp.float32),
        pltpu.VMEM_SHARED((_NP, _DH), jnp.float32),
        pltpu.SemaphoreType.DMA,
        pltpu.SemaphoreType.DMA,
    ],
)(_scat_body)


def _mm1_body(x_ref, w_ref, cnt_ref, z_ref, dinv_ref):
    i = pl.program_id(0)
    rows = i * _RB + lax.broadcasted_iota(jnp.int32, (_RB, 1), 0)
    valid = rows < _N
    xb = jnp.where(valid, x_ref[...], 0.0)
    h = jnp.dot(xb, w_ref[...], preferred_element_type=jnp.float32)
    deg = 1.0 + jnp.sum(cnt_ref[...], axis=0)[:, None]
    dinv = lax.rsqrt(deg)
    z_ref[...] = jnp.where(valid, h * dinv, 0.0)
    dinv_ref[...] = dinv


def _mm1(x, W1, cnt):
    return pl.pallas_call(
        _mm1_body,
        grid=(_GRID,),
        in_specs=[
            pl.BlockSpec((_RB, _DIN), lambda i: (i, 0)),
            pl.BlockSpec((_DIN, _DH), lambda i: (0, 0)),
            pl.BlockSpec((32, _RB), lambda i: (0, i)),
        ],
        out_specs=[
            pl.BlockSpec((_RB, _DH), lambda i: (i, 0)),
            pl.BlockSpec((_RB, 1), lambda i: (i, 0)),
        ],
        out_shape=[
            jax.ShapeDtypeStruct((_NP, _DH), jnp.float32),
            jax.ShapeDtypeStruct((_NP, 1), jnp.float32),
        ],
    )(x, W1, cnt)


def _mid_body(z1_ref, s1_ref, dinv_ref, b1_ref, z2_ref):
    i = pl.program_id(0)
    rows = i * _RB + lax.broadcasted_iota(jnp.int32, (_RB, 1), 0)
    valid = rows < _N
    dinv = dinv_ref[...]
    t = dinv * (z1_ref[...] + s1_ref[0] + s1_ref[1]) + b1_ref[...]
    z2_ref[...] = jnp.where(valid, dinv * jnp.maximum(t, 0.0), 0.0)


def _mid(z1, s1, dinv, b1):
    return pl.pallas_call(
        _mid_body,
        grid=(_GRID,),
        in_specs=[
            pl.BlockSpec((_RB, _DH), lambda i: (i, 0)),
            pl.BlockSpec((2, _RB, _DH), lambda i: (0, i, 0)),
            pl.BlockSpec((_RB, 1), lambda i: (i, 0)),
            pl.BlockSpec((1, _DH), lambda i: (0, 0)),
        ],
        out_specs=pl.BlockSpec((_RB, _DH), lambda i: (i, 0)),
        out_shape=jax.ShapeDtypeStruct((_NP, _DH), jnp.float32),
    )(z1, s1, dinv, b1)


def _mm2_body(z2_ref, s2_ref, dinv_ref, w_ref, b_ref, o_ref):
    dinv = dinv_ref[...]
    agg = dinv * (z2_ref[...] + s2_ref[0] + s2_ref[1])
    o_ref[...] = jnp.dot(agg, w_ref[...], preferred_element_type=jnp.float32) + b_ref[...]


def _mm2(z2, s2, dinv, W2, b2):
    return pl.pallas_call(
        _mm2_body,
        grid=(_GRID,),
        in_specs=[
            pl.BlockSpec((_RB, _DH), lambda i: (i, 0)),
            pl.BlockSpec((2, _RB, _DH), lambda i: (0, i, 0)),
            pl.BlockSpec((_RB, 1), lambda i: (i, 0)),
            pl.BlockSpec((_DH, _DOUT), lambda i: (0, 0)),
            pl.BlockSpec((1, _DOUT), lambda i: (0, 0)),
        ],
        out_specs=pl.BlockSpec((_RB, _DOUT), lambda i: (i, 0)),
        out_shape=jax.ShapeDtypeStruct((_N, _DOUT), jnp.float32),
    )(z2, s2, dinv, W2, b2)


def kernel(x, edge_index, W1, b1, W2, b2):
    src = edge_index[0].astype(jnp.int32)
    dst = edge_index[1].astype(jnp.int32)
    per_tile = _E // 32
    pad = jnp.full((2, 16, _EPT - per_tile), _N, jnp.int32)
    srcr = jnp.concatenate([src.reshape(2, 16, per_tile), pad], axis=2)
    srcr = srcr.reshape(2, 16, _NCK, _CH)
    dstr = jnp.concatenate([dst.reshape(2, 16, per_tile), pad], axis=2)
    dstr = dstr.reshape(2, 16, _NCK, _CH)

    cnt = _deg_kernel(dstr)
    z1, dinv = _mm1(x, W1, cnt)
    s1 = _scat_kernel(z1, srcr, dstr)
    z2 = _mid(z1, s1, dinv, b1.reshape(1, _DH))
    s2 = _scat_kernel(z2, srcr, dstr)
    return _mm2(z2, s2, dinv, W2, b2.reshape(1, _DOUT))


# speedup vs baseline: 1.7248x; 1.7248x over previous
import functools

import jax
import jax.numpy as jnp
from jax import lax
from jax.experimental import pallas as pl
from jax.experimental.pallas import tpu as pltpu
from jax.experimental.pallas import tpu_sc as plsc

_N = 10000
_E = 160000
_DIN = 256
_DH = 128
_DOUT = 256
_NP = 10240
_HALF = _DH // 2
_CH = 128
_EPT = 10240
_NCK = _EPT // _CH
_NB = 2
_RPT = _NP // 16
_RB = 1024
_GRID = _NP // _RB

_sc_mesh = plsc.VectorSubcoreMesh(core_axis_name="c", subcore_axis_name="s")


def _deg_body(dst_hbm, out_hbm, dstv, hist):
    c = lax.axis_index("c")
    s = lax.axis_index("s")
    half = _NCK // 2
    pltpu.sync_copy(dst_hbm.at[s, pl.ds(c * half, half)], dstv)

    def zb(i, carry):
        hist[pl.ds(i * 16, 16)] = jnp.zeros((16,), jnp.float32)
        return carry

    lax.fori_loop(0, _NP // 16, zb, 0)
    ones = jnp.full((16,), 1.0, jnp.float32)

    def hb(i, carry):
        for k in range(_CH // 16):
            idx = dstv[i, pl.ds(k * 16, 16)]
            plsc.addupdate_scatter(hist, [idx], ones)
        return carry

    lax.fori_loop(0, half, hb, 0)
    wid = c * 16 + s
    pltpu.sync_copy(hist, out_hbm.at[wid])


_deg_kernel = functools.partial(
    pl.kernel,
    out_type=jax.ShapeDtypeStruct((32, _NP), jnp.float32),
    mesh=_sc_mesh,
    compiler_params=pltpu.CompilerParams(needs_layout_passes=False),
    scratch_types=[
        pltpu.VMEM((_NCK // 2, _CH), jnp.int32),
        pltpu.VMEM((_NP,), jnp.float32),
    ],
)(_deg_body)


def _scat_body(z_hbm, src_hbm, dst_hbm, out_hbm, srcv, dstv, bufs, zs, acc, sems):
    c = lax.axis_index("c")
    s = lax.axis_index("s")
    pltpu.sync_copy(src_hbm.at[s], srcv)
    pltpu.sync_copy(dst_hbm.at[s], dstv)

    base = s * _RPT
    for k in range(_RPT // _CH):
        pltpu.sync_copy(z_hbm.at[c, pl.ds(base + k * _CH, _CH)],
                        zs.at[pl.ds(base + k * _CH, _CH)])

    def zb(i, carry):
        for k in range(_HALF // 16):
            bufs[0, i, pl.ds(k * 16, 16)] = jnp.zeros((16,), jnp.float32)
        return carry

    lax.fori_loop(0, _CH, zb, 0)
    for k in range(_RPT // _CH):
        pltpu.sync_copy(bufs.at[0], acc.at[pl.ds(base + k * _CH, _CH)])
    plsc.subcore_barrier()

    def start(k, b, sem):
        pltpu.async_copy(zs.at[srcv.at[k]], bufs.at[b], sem)

    def wait(b, sem):
        pltpu.make_async_copy(z_hbm.at[0, pl.ds(0, _CH)], bufs.at[b], sem).wait()

    def scat(k, b):
        pltpu.sync_copy(bufs.at[b], acc.at[dstv.at[k]], add=True)

    for b in range(_NB):
        start(b, b, sems.at[b])

    def body(j, carry):
        k = j * _NB
        for b in range(_NB):
            wait(b, sems.at[b])
            scat(k + b, b)

            @pl.when(k + b + _NB < _NCK)
            def _():
                start(k + b + _NB, b, sems.at[b])

        return carry

    lax.fori_loop(0, _NCK // _NB, body, 0)
    plsc.subcore_barrier()

    def wb(k, carry):
        pltpu.sync_copy(acc.at[pl.ds(base + k * _CH, _CH)], bufs.at[0])
        pltpu.sync_copy(bufs.at[0], out_hbm.at[c, pl.ds(base + k * _CH, _CH)])
        return carry

    lax.fori_loop(0, _RPT // _CH, wb, 0)


_scat_kernel = functools.partial(
    pl.kernel,
    out_type=jax.ShapeDtypeStruct((2, _NP, _HALF), jnp.float32),
    mesh=_sc_mesh,
    compiler_params=pltpu.CompilerParams(use_tc_tiling_on_sc=False),
    scratch_types=[
        pltpu.VMEM((_NCK, _CH), jnp.int32),
        pltpu.VMEM((_NCK, _CH), jnp.int32),
        pltpu.VMEM((_NB, _CH, _HALF), jnp.float32),
        pltpu.VMEM_SHARED((_NP, _HALF), jnp.float32),
        pltpu.VMEM_SHARED((_NP, _HALF), jnp.float32),
        pltpu.SemaphoreType.DMA((_NB,)),
    ],
)(_scat_body)


def _mm1_body(x_ref, w_ref, cnt_ref, z_ref, dinv_ref):
    i = pl.program_id(0)
    rows = i * _RB + lax.broadcasted_iota(jnp.int32, (_RB, 1), 0)
    valid = rows < _N
    xb = jnp.where(valid, x_ref[...], 0.0)
    h = jnp.dot(xb, w_ref[...], preferred_element_type=jnp.float32)
    deg = 1.0 + jnp.sum(cnt_ref[...], axis=0)[:, None]
    dinv = lax.rsqrt(deg)
    z = jnp.where(valid, h * dinv, 0.0)
    z_ref[0] = z[:, :_HALF]
    z_ref[1] = z[:, _HALF:]
    dinv_ref[...] = dinv


def _mm1(x, W1, cnt):
    return pl.pallas_call(
        _mm1_body,
        grid=(_GRID,),
        in_specs=[
            pl.BlockSpec((_RB, _DIN), lambda i: (i, 0)),
            pl.BlockSpec((_DIN, _DH), lambda i: (0, 0)),
            pl.BlockSpec((32, _RB), lambda i: (0, i)),
        ],
        out_specs=[
            pl.BlockSpec((2, _RB, _HALF), lambda i: (0, i, 0)),
            pl.BlockSpec((_RB, 1), lambda i: (i, 0)),
        ],
        out_shape=[
            jax.ShapeDtypeStruct((2, _NP, _HALF), jnp.float32),
            jax.ShapeDtypeStruct((_NP, 1), jnp.float32),
        ],
    )(x, W1, cnt)


def _mid_body(z1_ref, s1_ref, dinv_ref, b1_ref, z2_ref):
    i = pl.program_id(0)
    rows = i * _RB + lax.broadcasted_iota(jnp.int32, (_RB, 1), 0)
    valid = rows < _N
    dinv = dinv_ref[...]
    for c in range(2):
        t = dinv * (z1_ref[c] + s1_ref[c]) + b1_ref[c][None, :]
        z2_ref[c] = jnp.where(valid, dinv * jnp.maximum(t, 0.0), 0.0)


def _mid(z1, s1, dinv, b1):
    return pl.pallas_call(
        _mid_body,
        grid=(_GRID,),
        in_specs=[
            pl.BlockSpec((2, _RB, _HALF), lambda i: (0, i, 0)),
            pl.BlockSpec((2, _RB, _HALF), lambda i: (0, i, 0)),
            pl.BlockSpec((_RB, 1), lambda i: (i, 0)),
            pl.BlockSpec((2, _HALF), lambda i: (0, 0)),
        ],
        out_specs=pl.BlockSpec((2, _RB, _HALF), lambda i: (0, i, 0)),
        out_shape=jax.ShapeDtypeStruct((2, _NP, _HALF), jnp.float32),
    )(z1, s1, dinv, b1)


def _mm2_body(z2_ref, s2_ref, dinv_ref, w_ref, b_ref, o_ref):
    dinv = dinv_ref[...]
    a0 = dinv * (z2_ref[0] + s2_ref[0])
    a1 = dinv * (z2_ref[1] + s2_ref[1])
    agg = jnp.concatenate([a0, a1], axis=1)
    o_ref[...] = jnp.dot(agg, w_ref[...], preferred_element_type=jnp.float32) + b_ref[...]


def _mm2(z2, s2, dinv, W2, b2):
    return pl.pallas_call(
        _mm2_body,
        grid=(_GRID,),
        in_specs=[
            pl.BlockSpec((2, _RB, _HALF), lambda i: (0, i, 0)),
            pl.BlockSpec((2, _RB, _HALF), lambda i: (0, i, 0)),
            pl.BlockSpec((_RB, 1), lambda i: (i, 0)),
            pl.BlockSpec((_DH, _DOUT), lambda i: (0, 0)),
            pl.BlockSpec((1, _DOUT), lambda i: (0, 0)),
        ],
        out_specs=pl.BlockSpec((_RB, _DOUT), lambda i: (i, 0)),
        out_shape=jax.ShapeDtypeStruct((_N, _DOUT), jnp.float32),
    )(z2, s2, dinv, W2, b2)


def kernel(x, edge_index, W1, b1, W2, b2):
    src = edge_index[0].astype(jnp.int32)
    dst = edge_index[1].astype(jnp.int32)
    per_tile = _E // 16
    pad = jnp.full((16, _EPT - per_tile), _N, jnp.int32)
    srcr = jnp.concatenate([src.reshape(16, per_tile), pad], axis=1)
    srcr = srcr.reshape(16, _NCK, _CH)
    dstr = jnp.concatenate([dst.reshape(16, per_tile), pad], axis=1)
    dstr = dstr.reshape(16, _NCK, _CH)

    cnt = _deg_kernel(dstr)
    z1, dinv = _mm1(x, W1, cnt)
    s1 = _scat_kernel(z1, srcr, dstr)
    z2 = _mid(z1, s1, dinv, b1.reshape(2, _HALF))
    s2 = _scat_kernel(z2, srcr, dstr)
    return _mm2(z2, s2, dinv, W2, b2.reshape(1, _DOUT))


# speedup vs baseline: 1.8533x; 1.0745x over previous
import functools

import jax
import jax.numpy as jnp
from jax import lax
from jax.experimental import pallas as pl
from jax.experimental.pallas import tpu as pltpu
from jax.experimental.pallas import tpu_sc as plsc

_N = 10000
_E = 160000
_DIN = 256
_DH = 128
_DOUT = 256
_NP = 10240
_HALF = _DH // 2
_CH = 80
_NCK = 125
_RPT = _NP // 16
_RB = 1024
_GRID = _NP // _RB

_sc_mesh = plsc.VectorSubcoreMesh(core_axis_name="c", subcore_axis_name="s")


def _deg_body(dst_hbm, out_hbm, dstv, hist):
    c = lax.axis_index("c")
    s = lax.axis_index("s")
    pltpu.sync_copy(dst_hbm.at[s], dstv)

    def zb(i, carry):
        hist[pl.ds(i * 16, 16)] = jnp.zeros((16,), jnp.float32)
        return carry

    lax.fori_loop(0, _NP // 16, zb, 0)
    ones = jnp.full((16,), 1.0, jnp.float32)

    def hb(i, carry):
        for k in range(_CH // 16):
            idx = dstv[i, pl.ds(k * 16, 16)]
            plsc.addupdate_scatter(hist, [idx], ones)
        return carry

    lax.fori_loop(c * 62, 62 + c * 63, hb, 0)
    wid = c * 16 + s
    pltpu.sync_copy(hist, out_hbm.at[wid])


_deg_kernel = functools.partial(
    pl.kernel,
    out_type=jax.ShapeDtypeStruct((32, _NP), jnp.float32),
    mesh=_sc_mesh,
    compiler_params=pltpu.CompilerParams(needs_layout_passes=False),
    scratch_types=[
        pltpu.VMEM((_NCK, _CH), jnp.int32),
        pltpu.VMEM((_NP,), jnp.float32),
    ],
)(_deg_body)


def _agg_body(z_hbm, dinv_hbm, bb_hbm, src_hbm, dst_hbm, out_hbm,
              srcv, dstv, b0, b1, zbuf, dinv_v, bbv, zs, acc, sems):
    c = lax.axis_index("c")
    s = lax.axis_index("s")
    base = s * _RPT
    pltpu.sync_copy(src_hbm.at[s], srcv)
    pltpu.sync_copy(dst_hbm.at[s], dstv)
    pltpu.sync_copy(dinv_hbm.at[pl.ds(base, _RPT)], dinv_v)
    pltpu.sync_copy(bb_hbm.at[c], bbv)

    def zb(i, carry):
        for k in range(_HALF // 16):
            zbuf[i, pl.ds(k * 16, 16)] = jnp.zeros((16,), jnp.float32)
        return carry

    lax.fori_loop(0, _CH, zb, 0)
    for k in range(_RPT // _CH):
        pltpu.sync_copy(z_hbm.at[c, pl.ds(base + k * _CH, _CH)],
                        zs.at[pl.ds(base + k * _CH, _CH)])
        pltpu.sync_copy(zbuf, acc.at[pl.ds(base + k * _CH, _CH)])
    plsc.subcore_barrier()

    def start(k, buf, sem):
        pltpu.async_copy(zs.at[srcv.at[k]], buf, sem)

    def wait(buf, sem):
        pltpu.make_async_copy(z_hbm.at[0, pl.ds(0, _CH)], buf, sem).wait()

    def scat(k, buf):
        pltpu.sync_copy(buf, acc.at[dstv.at[k]], add=True)

    def agg_pass():
        start(0, b0, sems.at[0])
        start(1, b1, sems.at[1])

        def body(j, carry):
            k = j * 2
            wait(b0, sems.at[0])
            scat(k, b0)

            @pl.when(k + 2 < _NCK)
            def _():
                start(k + 2, b0, sems.at[0])

            wait(b1, sems.at[1])
            scat(k + 1, b1)

            @pl.when(k + 3 < _NCK)
            def _():
                start(k + 3, b1, sems.at[1])

            return carry

        lax.fori_loop(0, (_NCK - 1) // 2, body, 0)
        wait(b0, sems.at[0])
        scat(_NCK - 1, b0)

    agg_pass()
    plsc.subcore_barrier()

    def epi(q, final):
        rb = base + q * _CH
        pltpu.sync_copy(acc.at[pl.ds(rb, _CH)], b0)
        pltpu.sync_copy(zs.at[pl.ds(rb, _CH)], b1)

        def rowfn(r, carry):
            db = dinv_v[q * _CH + r]
            for g in range(_HALF // 16):
                zv = b1[r, pl.ds(g * 16, 16)]
                av = b0[r, pl.ds(g * 16, 16)]
                if final:
                    b1[r, pl.ds(g * 16, 16)] = db * (zv + av)
                else:
                    bv = bbv[pl.ds(g * 16, 16)]
                    t = db * (zv + av) + bv
                    b1[r, pl.ds(g * 16, 16)] = db * jnp.maximum(t, 0.0)
            return carry

        lax.fori_loop(0, _CH, rowfn, 0)
        if final:
            pltpu.sync_copy(b1, out_hbm.at[c, pl.ds(rb, _CH)])
        else:
            pltpu.sync_copy(b1, zs.at[pl.ds(rb, _CH)])
            pltpu.sync_copy(zbuf, acc.at[pl.ds(rb, _CH)])

    for q in range(_RPT // _CH):
        epi(q, final=False)
    plsc.subcore_barrier()

    agg_pass()
    plsc.subcore_barrier()

    for q in range(_RPT // _CH):
        epi(q, final=True)


_agg_kernel = functools.partial(
    pl.kernel,
    out_type=jax.ShapeDtypeStruct((2, _NP, _HALF), jnp.float32),
    mesh=_sc_mesh,
    compiler_params=pltpu.CompilerParams(use_tc_tiling_on_sc=False),
    scratch_types=[
        pltpu.VMEM((_NCK, _CH), jnp.int32),
        pltpu.VMEM((_NCK, _CH), jnp.int32),
        pltpu.VMEM((_CH, _HALF), jnp.float32),
        pltpu.VMEM((_CH, _HALF), jnp.float32),
        pltpu.VMEM((_CH, _HALF), jnp.float32),
        pltpu.VMEM((_RPT, 16), jnp.float32),
        pltpu.VMEM((_HALF,), jnp.float32),
        pltpu.VMEM_SHARED((_NP, _HALF), jnp.float32),
        pltpu.VMEM_SHARED((_NP, _HALF), jnp.float32),
        pltpu.SemaphoreType.DMA((2,)),
    ],
)(_agg_body)


def _mm1_body(x_ref, w_ref, cnt_ref, z_ref, dinv_ref):
    i = pl.program_id(0)
    rows = i * _RB + lax.broadcasted_iota(jnp.int32, (_RB, 1), 0)
    valid = rows < _N
    xb = jnp.where(valid, x_ref[...], 0.0)
    h = jnp.dot(xb, w_ref[...], preferred_element_type=jnp.float32)
    deg = 1.0 + jnp.sum(cnt_ref[...], axis=0)[:, None]
    dinv = lax.rsqrt(deg)
    z = jnp.where(valid, h * dinv, 0.0)
    z_ref[0] = z[:, :_HALF]
    z_ref[1] = z[:, _HALF:]
    dinv_ref[...] = jnp.broadcast_to(dinv, (_RB, 16))


def _mm1(x, W1, cnt):
    return pl.pallas_call(
        _mm1_body,
        grid=(_GRID,),
        in_specs=[
            pl.BlockSpec((_RB, _DIN), lambda i: (i, 0)),
            pl.BlockSpec((_DIN, _DH), lambda i: (0, 0)),
            pl.BlockSpec((32, _RB), lambda i: (0, i)),
        ],
        out_specs=[
            pl.BlockSpec((2, _RB, _HALF), lambda i: (0, i, 0)),
            pl.BlockSpec((_RB, 16), lambda i: (i, 0)),
        ],
        out_shape=[
            jax.ShapeDtypeStruct((2, _NP, _HALF), jnp.float32),
            jax.ShapeDtypeStruct((_NP, 16), jnp.float32),
        ],
    )(x, W1, cnt)


def _mm2_body(sf_ref, w_ref, b_ref, o_ref):
    agg = jnp.concatenate([sf_ref[0], sf_ref[1]], axis=1)
    o_ref[...] = jnp.dot(agg, w_ref[...], preferred_element_type=jnp.float32) + b_ref[...]


def _mm2(sf, W2, b2):
    return pl.pallas_call(
        _mm2_body,
        grid=(_GRID,),
        in_specs=[
            pl.BlockSpec((2, _RB, _HALF), lambda i: (0, i, 0)),
            pl.BlockSpec((_DH, _DOUT), lambda i: (0, 0)),
            pl.BlockSpec((1, _DOUT), lambda i: (0, 0)),
        ],
        out_specs=pl.BlockSpec((_RB, _DOUT), lambda i: (i, 0)),
        out_shape=jax.ShapeDtypeStruct((_N, _DOUT), jnp.float32),
    )(sf, W2, b2)


def kernel(x, edge_index, W1, b1, W2, b2):
    ei = edge_index.astype(jnp.int32).reshape(2, 16, _NCK, _CH)
    srcr, dstr = ei[0], ei[1]

    cnt = _deg_kernel(dstr)
    z1, dinv = _mm1(x, W1, cnt)
    sf = _agg_kernel(z1, dinv, b1.reshape(2, _HALF), srcr, dstr)
    return _mm2(sf, W2, b2.reshape(1, _DOUT))
